# baseline (speedup 1.0000x reference)
import functools

import jax
import jax.numpy as jnp
from jax import lax
from jax.experimental import pallas as pl
from jax.experimental.pallas import tpu as pltpu
from jax.experimental.pallas import tpu_sc as plsc

_NC = 2
_NS = 16
_NW = _NC * _NS
_NBUF = 3


def _sc_gather_sum(table, idx3, K, NCH_EFF, NCHUNKS):
    R, HD = table.shape
    NW, NCH_PAD, CBK = idx3.shape
    CB = CBK // K

    mesh = plsc.VectorSubcoreMesh(core_axis_name="c", subcore_axis_name="s")

    @functools.partial(
        pl.kernel,
        out_type=jax.ShapeDtypeStruct((R, HD), jnp.float32),
        mesh=mesh,
        scratch_types=[
            pltpu.VMEM((NCH_PAD, CBK), jnp.int32),
            [pltpu.VMEM((CBK, HD), jnp.float32) for _ in range(_NBUF)],
            [pltpu.VMEM((CB, HD), jnp.float32) for _ in range(_NBUF)],
            [pltpu.SemaphoreType.DMA for _ in range(_NBUF)],
            [pltpu.SemaphoreType.DMA for _ in range(_NBUF)],
        ],
    )
    def k(tab_hbm, idx_hbm, out_hbm, idx_v, gbufs, obufs, gsems, osems):
        wid = lax.axis_index("s") * _NC + lax.axis_index("c")
        pltpu.sync_copy(idx_hbm.at[wid], idx_v)

        def row0_of(c):
            return pl.multiple_of(
                jnp.minimum(wid + NW * c, NCHUNKS - 1) * CB, CB)

        for b in range(_NBUF):
            pltpu.make_async_copy(
                tab_hbm.at[idx_v.at[b]], gbufs[b], gsems[b]).start()

        def step(i, carry):
            for b in range(_NBUF):
                c = i * _NBUF + b
                gb, ob = gbufs[b], obufs[b]
                gs, os_ = gsems[b], osems[b]
                row0 = row0_of(c)
                pltpu.make_async_copy(tab_hbm.at[idx_v.at[c]], gb, gs).wait()
                @pl.when(c >= _NBUF)
                def _():
                    pltpu.make_async_copy(
                        ob, out_hbm.at[pl.ds(0, CB)], os_).wait()

                def row(r, rc):
                    rb = r * K
                    for h in range(HD // 16):
                        s = pl.ds(h * 16, 16)
                        a = gb[rb, s]
                        for kk in range(1, K):
                            a = a + gb[rb + kk, s]
                        ob[r, s] = a
                    return rc

                lax.fori_loop(0, CB, row, 0)
                @pl.when(c + _NBUF < NCH_EFF)
                def _():
                    pltpu.make_async_copy(
                        tab_hbm.at[idx_v.at[c + _NBUF]], gb, gs).start()
                pltpu.make_async_copy(
                    ob, out_hbm.at[pl.ds(row0, CB)], os_).start()
            return carry

        lax.fori_loop(0, NCH_EFF // _NBUF, step, 0)
        for b in range(_NBUF):
            pltpu.make_async_copy(
                obufs[b], out_hbm.at[pl.ds(0, CB)], osems[b]).wait()

    return k(table, idx3)


def _tc_combine(Hf, G, W, b2, K):
    N, DHD = Hf.shape
    D, _, HD = G.shape
    BN = 1000
    scale = 1.0 / K

    def body(h_ref, g_ref, w_ref, b_ref, o_ref):
        w = w_ref[...]
        bb = b_ref[...]
        for d in range(D):
            g = g_ref[d] * scale
            m = lax.dot_general(g, w, (((1,), (1,)), ((), ())),
                                preferred_element_type=jnp.float32)
            o_ref[:, d * HD:(d + 1) * HD] = h_ref[:, d * HD:(d + 1) * HD] + m + bb

    return pl.pallas_call(
        body,
        grid=(N // BN,),
        in_specs=[
            pl.BlockSpec((BN, DHD), lambda i: (i, 0)),
            pl.BlockSpec((D, BN, HD), lambda i: (0, i, 0)),
            pl.BlockSpec((HD, HD), lambda i: (0, 0)),
            pl.BlockSpec((1, HD), lambda i: (0, 0)),
        ],
        out_specs=pl.BlockSpec((BN, DHD), lambda i: (i, 0)),
        out_shape=jax.ShapeDtypeStruct((N, DHD), jnp.float32),
    )(Hf, G, W, b2)


def kernel(H, knn_idx, W, b):
    N, D, HD = H.shape
    K = knn_idx.shape[-1]
    R = N * D
    CB = 8
    NCHUNKS = R // CB
    NCH_EFF = -(-NCHUNKS // _NW)
    NCH_EFF = -(-NCH_EFF // _NBUF) * _NBUF
    NCH_PAD = -(-(NCH_EFF) // 8) * 8

    H2 = H.reshape(R, HD)
    offs = jnp.arange(D, dtype=jnp.int32)[:, None, None]
    idx_chunks = (knn_idx * D + offs).reshape(NCHUNKS, CB * K)
    cids = jnp.minimum(
        jnp.arange(_NW, dtype=jnp.int32)[:, None]
        + _NW * jnp.arange(NCH_PAD, dtype=jnp.int32)[None, :],
        NCHUNKS - 1)
    idx3 = idx_chunks[cids]

    G = _sc_gather_sum(H2, idx3, K, NCH_EFF, NCHUNKS)
    out = _tc_combine(H.reshape(N, D * HD), G.reshape(D, N, HD),
                      W, b.reshape(1, HD), K)
    return out.reshape(N, D, HD)


# speedup vs baseline: 1.4085x; 1.4085x over previous
import functools

import jax
import jax.numpy as jnp
from jax import lax
from jax.experimental import pallas as pl
from jax.experimental.pallas import tpu as pltpu
from jax.experimental.pallas import tpu_sc as plsc

_NC = 2
_NS = 16
_NW = _NC * _NS
_NBUF = 3


def _sc_gather_sum(table, idx3, K, NCH_EFF, NCHUNKS):
    R, HD = table.shape
    NW, NCH_PAD, CBK = idx3.shape
    CB = CBK // K

    mesh = plsc.VectorSubcoreMesh(core_axis_name="c", subcore_axis_name="s")

    @functools.partial(
        pl.kernel,
        out_type=jax.ShapeDtypeStruct((R, HD), jnp.float32),
        mesh=mesh,
        scratch_types=[
            pltpu.VMEM((NCH_PAD, CBK), jnp.int32),
            [pltpu.VMEM((CBK, HD), jnp.float32) for _ in range(_NBUF)],
            [pltpu.VMEM((CB, HD), jnp.float32) for _ in range(_NBUF)],
            [pltpu.SemaphoreType.DMA for _ in range(_NBUF)],
            [pltpu.SemaphoreType.DMA for _ in range(_NBUF)],
        ],
    )
    def k(tab_hbm, idx_hbm, out_hbm, idx_v, gbufs, obufs, gsems, osems):
        wid = lax.axis_index("s") * _NC + lax.axis_index("c")
        pltpu.sync_copy(idx_hbm.at[wid], idx_v)

        def row0_of(c):
            return pl.multiple_of(
                jnp.minimum(wid + NW * c, NCHUNKS - 1) * CB, CB)

        for b in range(_NBUF):
            pltpu.make_async_copy(
                tab_hbm.at[idx_v.at[b]], gbufs[b], gsems[b]).start()

        def step(i, carry):
            for b in range(_NBUF):
                c = i * _NBUF + b
                gb, ob = gbufs[b], obufs[b]
                gs, os_ = gsems[b], osems[b]
                row0 = row0_of(c)
                pltpu.make_async_copy(tab_hbm.at[idx_v.at[c]], gb, gs).wait()
                @pl.when(c >= _NBUF)
                def _():
                    pltpu.make_async_copy(
                        ob, out_hbm.at[pl.ds(0, CB)], os_).wait()

                def row(r, rc):
                    rb = r * K
                    acc = [gb[rb, pl.ds(h * 16, 16)]
                           for h in range(HD // 16)]
                    for kk in range(1, K):
                        for h in range(HD // 16):
                            acc[h] += gb[rb + kk, pl.ds(h * 16, 16)]
                    for h in range(HD // 16):
                        ob[r, pl.ds(h * 16, 16)] = acc[h]
                    return rc

                lax.fori_loop(0, CB, row, 0)
                @pl.when(c + _NBUF < NCH_EFF)
                def _():
                    pltpu.make_async_copy(
                        tab_hbm.at[idx_v.at[c + _NBUF]], gb, gs).start()
                pltpu.make_async_copy(
                    ob, out_hbm.at[pl.ds(row0, CB)], os_).start()
            return carry

        lax.fori_loop(0, NCH_EFF // _NBUF, step, 0)
        for b in range(_NBUF):
            pltpu.make_async_copy(
                obufs[b], out_hbm.at[pl.ds(0, CB)], osems[b]).wait()

    return k(table, idx3)


def _tc_combine(Hf, G, W, b2, K):
    N, DHD = Hf.shape
    D, _, HD = G.shape
    BN = 1000
    scale = 1.0 / K

    def body(h_ref, g_ref, w_ref, b_ref, o_ref):
        w = w_ref[...]
        bb = b_ref[...]
        for d in range(D):
            g = g_ref[d] * scale
            m = lax.dot_general(g, w, (((1,), (1,)), ((), ())),
                                preferred_element_type=jnp.float32)
            o_ref[:, d * HD:(d + 1) * HD] = h_ref[:, d * HD:(d + 1) * HD] + m + bb

    return pl.pallas_call(
        body,
        grid=(N // BN,),
        in_specs=[
            pl.BlockSpec((BN, DHD), lambda i: (i, 0)),
            pl.BlockSpec((D, BN, HD), lambda i: (0, i, 0)),
            pl.BlockSpec((HD, HD), lambda i: (0, 0)),
            pl.BlockSpec((1, HD), lambda i: (0, 0)),
        ],
        out_specs=pl.BlockSpec((BN, DHD), lambda i: (i, 0)),
        out_shape=jax.ShapeDtypeStruct((N, DHD), jnp.float32),
    )(Hf, G, W, b2)


def kernel(H, knn_idx, W, b):
    N, D, HD = H.shape
    K = knn_idx.shape[-1]
    R = N * D
    CB = 8
    NCHUNKS = R // CB
    NCH_EFF = -(-NCHUNKS // _NW)
    NCH_EFF = -(-NCH_EFF // _NBUF) * _NBUF
    NCH_PAD = -(-(NCH_EFF) // 8) * 8

    H2 = H.reshape(R, HD)
    offs = jnp.arange(D, dtype=jnp.int32)[:, None, None]
    idx_chunks = (knn_idx * D + offs).reshape(NCHUNKS, CB * K)
    cids = jnp.minimum(
        jnp.arange(_NW, dtype=jnp.int32)[:, None]
        + _NW * jnp.arange(NCH_PAD, dtype=jnp.int32)[None, :],
        NCHUNKS - 1)
    idx3 = idx_chunks[cids]

    G = _sc_gather_sum(H2, idx3, K, NCH_EFF, NCHUNKS)
    out = _tc_combine(H.reshape(N, D * HD), G.reshape(D, N, HD),
                      W, b.reshape(1, HD), K)
    return out.reshape(N, D, HD)


# speedup vs baseline: 1.5138x; 1.0748x over previous
import functools

import jax
import jax.numpy as jnp
from jax import lax
from jax.experimental import pallas as pl
from jax.experimental.pallas import tpu as pltpu
from jax.experimental.pallas import tpu_sc as plsc

_NC = 2
_NS = 16
_NW = _NC * _NS
_NBUF = 3


def _sc_gather_sum(table, idx3, K, NCH_EFF, NCHUNKS):
    R, HD = table.shape
    NCH_PAD, NW, CBK = idx3.shape
    CB = CBK // K

    mesh = plsc.VectorSubcoreMesh(core_axis_name="c", subcore_axis_name="s")

    @functools.partial(
        pl.kernel,
        out_type=jax.ShapeDtypeStruct((R, HD), jnp.float32),
        mesh=mesh,
        scratch_types=[
            pltpu.VMEM((NCH_PAD, CBK), jnp.int32),
            [pltpu.VMEM((CBK, HD), jnp.float32) for _ in range(_NBUF)],
            [pltpu.VMEM((CB, HD), jnp.float32) for _ in range(_NBUF)],
            [pltpu.SemaphoreType.DMA for _ in range(_NBUF)],
            [pltpu.SemaphoreType.DMA for _ in range(_NBUF)],
        ],
    )
    def k(tab_hbm, idx_hbm, out_hbm, idx_v, gbufs, obufs, gsems, osems):
        wid = lax.axis_index("s") * _NC + lax.axis_index("c")
        pltpu.sync_copy(idx_hbm.at[:, wid], idx_v)

        def row0_of(c):
            return pl.multiple_of(
                jnp.minimum(c * NW + wid, NCHUNKS - 1) * CB, CB)

        for b in range(_NBUF):
            pltpu.make_async_copy(
                tab_hbm.at[idx_v.at[b]], gbufs[b], gsems[b]).start()

        def step(i, carry):
            for b in range(_NBUF):
                c = i * _NBUF + b
                gb, ob = gbufs[b], obufs[b]
                gs, os_ = gsems[b], osems[b]
                row0 = row0_of(c)
                pltpu.make_async_copy(tab_hbm.at[idx_v.at[c]], gb, gs).wait()
                @pl.when(c >= _NBUF)
                def _():
                    pltpu.make_async_copy(
                        ob, out_hbm.at[pl.ds(0, CB)], os_).wait()

                def row(r, rc):
                    rb = r * K
                    acc = [gb[rb, pl.ds(h * 16, 16)]
                           for h in range(HD // 16)]
                    for kk in range(1, K):
                        for h in range(HD // 16):
                            acc[h] += gb[rb + kk, pl.ds(h * 16, 16)]
                    for h in range(HD // 16):
                        ob[r, pl.ds(h * 16, 16)] = acc[h]
                    return rc

                lax.fori_loop(0, CB, row, 0)
                @pl.when(c + _NBUF < NCH_EFF)
                def _():
                    pltpu.make_async_copy(
                        tab_hbm.at[idx_v.at[c + _NBUF]], gb, gs).start()
                pltpu.make_async_copy(
                    ob, out_hbm.at[pl.ds(row0, CB)], os_).start()
            return carry

        lax.fori_loop(0, NCH_EFF // _NBUF, step, 0)
        for b in range(_NBUF):
            pltpu.make_async_copy(
                obufs[b], out_hbm.at[pl.ds(0, CB)], osems[b]).wait()

    return k(table, idx3)


def _tc_combine(Hf, G, W, b2, K):
    N, DHD = Hf.shape
    D, _, HD = G.shape
    BN = 1000
    scale = 1.0 / K

    def body(h_ref, g_ref, w_ref, b_ref, o_ref):
        w = w_ref[...]
        bb = b_ref[...]
        for d in range(D):
            g = g_ref[d] * scale
            m = lax.dot_general(g, w, (((1,), (1,)), ((), ())),
                                preferred_element_type=jnp.float32)
            o_ref[:, d * HD:(d + 1) * HD] = h_ref[:, d * HD:(d + 1) * HD] + m + bb

    return pl.pallas_call(
        body,
        grid=(N // BN,),
        in_specs=[
            pl.BlockSpec((BN, DHD), lambda i: (i, 0)),
            pl.BlockSpec((D, BN, HD), lambda i: (0, i, 0)),
            pl.BlockSpec((HD, HD), lambda i: (0, 0)),
            pl.BlockSpec((1, HD), lambda i: (0, 0)),
        ],
        out_specs=pl.BlockSpec((BN, DHD), lambda i: (i, 0)),
        out_shape=jax.ShapeDtypeStruct((N, DHD), jnp.float32),
    )(Hf, G, W, b2)


def kernel(H, knn_idx, W, b):
    N, D, HD = H.shape
    K = knn_idx.shape[-1]
    R = N * D
    CB = 8
    NCHUNKS = R // CB
    NCH_EFF = -(-NCHUNKS // _NW)
    NCH_EFF = -(-NCH_EFF // _NBUF) * _NBUF
    NCH_PAD = -(-(NCH_EFF) // 8) * 8

    H2 = H.reshape(R, HD)
    offs = jnp.arange(D, dtype=jnp.int32)[:, None, None]
    idx_chunks = (knn_idx * D + offs).reshape(NCHUNKS, CB * K)
    pad = NCH_PAD * _NW - NCHUNKS
    idx3 = jnp.concatenate(
        [idx_chunks,
         jnp.broadcast_to(idx_chunks[-1:], (pad, CB * K))],
        axis=0).reshape(NCH_PAD, _NW, CB * K)

    G = _sc_gather_sum(H2, idx3, K, NCH_EFF, NCHUNKS)
    out = _tc_combine(H.reshape(N, D * HD), G.reshape(D, N, HD),
                      W, b.reshape(1, HD), K)
    return out.reshape(N, D, HD)
